# baseline (speedup 1.0000x reference)
import jax
import jax.numpy as jnp
from jax import lax
from jax.experimental import pallas as pl
from jax.experimental.pallas import tpu as pltpu
from jax.experimental.pallas import tpu_sc as plsc

NUM_V = 50000
NUM_C = 20000
P = 32

L = 16
NS = 16
GC = 16
SLOTS = GC * P
CH = SLOTS // 128
NG = NUM_C // GC
GPW = -(-NG // NS)
TW = 16
AW = 8
VPAD = 51200
VT = VPAD // NS
VC = VT // 2


def _sc_body(xh, yh, zh, mh, wh, ch, idxh, ixh,
             ox, oy, oz,
             table_hs, acc_s,
             pbx, pby, pbz, pbm, pbw, pbc, tb, ab,
             idx_v, rows_v, init_v, delta_v, sem):
    wid = lax.axis_index("s")
    lids = lax.iota(jnp.int32, L)
    lid32 = lids * P
    lid96 = lids * (P * 3)
    cols = [jnp.full((L,), c, jnp.int32) for c in range(TW)]
    zero16 = jnp.zeros((L,), jnp.float32)
    o0 = wid * VT

    planes = (pbx, pby, pbz, pbm, pbw, pbc)

    for k in range(2):
        base = o0 + k * VC
        bsl = pl.ds(base, VC)
        for src, dst in zip((xh, yh, zh, mh, wh, ch), planes):
            pltpu.sync_copy(src.at[bsl], dst)

        def pack_body(i, carry):
            rows = i * jnp.int32(L) + lids
            isl = pl.ds(i * jnp.int32(L), L)
            for c in range(6):
                v = planes[c][isl]
                plsc.store_scatter(tb, [rows, cols[c]], v)
                if c < 3:
                    plsc.store_scatter(ab, [rows, cols[c]], v)
            for c in range(3, AW):
                plsc.store_scatter(ab, [rows, cols[c]], zero16)
            return carry

        lax.fori_loop(jnp.int32(0), jnp.int32(VC // L), pack_body,
                      jnp.int32(0))
        pltpu.sync_copy(tb, table_hs.at[bsl])
        pltpu.sync_copy(ab, acc_s.at[bsl])

    for q in range(SLOTS // L):
        sv = lids + q * L
        jv = lax.shift_right_logical(sv, jnp.int32(7))
        rv = lax.bitwise_and(sv, jnp.int32(127))
        for c in range(3, AW):
            plsc.store_scatter(delta_v, [jv, rv, cols[c]], zero16)

    plsc.subcore_barrier()

    def group_body(g, carry):
        gi = g * jnp.int32(NS) + wid

        @pl.when(gi < NG)
        def _():
            pltpu.sync_copy(idxh.at[pl.ds(gi * jnp.int32(CH), CH)], idx_v)
            descs = [pltpu.async_copy(table_hs.at[idx_v.at[jnp.int32(j)]],
                                      rows_v.at[jnp.int32(j)], sem)
                     for j in range(CH)]
            pltpu.sync_copy(ixh.at[pl.ds(gi * jnp.int32(CH * 3), CH * 3)],
                            init_v)
            for d in descs:
                d.wait()

            msum = zero16
            wx = zero16
            wy = zero16
            wz = zero16
            for p in range(P):
                sv = lid32 + p
                jv = lax.shift_right_logical(sv, jnp.int32(7))
                rv = lax.bitwise_and(sv, jnp.int32(127))
                x = plsc.load_gather(rows_v, [jv, rv, cols[0]])
                y = plsc.load_gather(rows_v, [jv, rv, cols[1]])
                z = plsc.load_gather(rows_v, [jv, rv, cols[2]])
                m = plsc.load_gather(rows_v, [jv, rv, cols[3]])
                msum = msum + m
                wx = wx + m * x
                wy = wy + m * y
                wz = wz + m * z
            cx = wx / msum
            cy = wy / msum
            cz = wz / msum

            for p in range(P):
                sv = lid32 + p
                jv = lax.shift_right_logical(sv, jnp.int32(7))
                rv = lax.bitwise_and(sv, jnp.int32(127))
                x = plsc.load_gather(rows_v, [jv, rv, cols[0]])
                y = plsc.load_gather(rows_v, [jv, rv, cols[1]])
                z = plsc.load_gather(rows_v, [jv, rv, cols[2]])
                w = plsc.load_gather(rows_v, [jv, rv, cols[4]])
                cm = plsc.load_gather(rows_v, [jv, rv, cols[5]])
                ov = lid96 + 3 * p
                irow = lax.shift_right_logical(ov, jnp.int32(7))
                icol = lax.bitwise_and(ov, jnp.int32(127))
                ix = plsc.load_gather(init_v, [irow, icol])
                ov1 = ov + 1
                iy = plsc.load_gather(
                    init_v, [lax.shift_right_logical(ov1, jnp.int32(7)),
                             lax.bitwise_and(ov1, jnp.int32(127))])
                ov2 = ov + 2
                iz = plsc.load_gather(
                    init_v, [lax.shift_right_logical(ov2, jnp.int32(7)),
                             lax.bitwise_and(ov2, jnp.int32(127))])
                s = w / cm
                plsc.store_scatter(delta_v, [jv, rv, cols[0]], s * (ix - x + cx))
                plsc.store_scatter(delta_v, [jv, rv, cols[1]], s * (iy - y + cy))
                plsc.store_scatter(delta_v, [jv, rv, cols[2]], s * (iz - z + cz))

            for j in range(CH):
                pltpu.sync_copy(delta_v.at[jnp.int32(j)],
                                acc_s.at[idx_v.at[jnp.int32(j)]], add=True)

        return carry

    lax.fori_loop(jnp.int32(0), jnp.int32(GPW), group_body, jnp.int32(0))
    plsc.subcore_barrier()

    for k in range(2):
        base = o0 + k * VC
        bsl = pl.ds(base, VC)
        pltpu.sync_copy(acc_s.at[bsl], ab)

        def unpack_body(i, carry):
            rows = i * jnp.int32(L) + lids
            isl = pl.ds(i * jnp.int32(L), L)
            for c, pb in ((0, pbx), (1, pby), (2, pbz)):
                pb[isl] = plsc.load_gather(ab, [rows, cols[c]])
            return carry

        lax.fori_loop(jnp.int32(0), jnp.int32(VC // L), unpack_body,
                      jnp.int32(0))
        pltpu.sync_copy(pbx, ox.at[bsl])
        pltpu.sync_copy(pby, oy.at[bsl])
        pltpu.sync_copy(pbz, oz.at[bsl])


@jax.jit
def _sc_call(xh, yh, zh, mh, wh, ch, idxh, ixh):
    mesh = plsc.VectorSubcoreMesh(core_axis_name="c", subcore_axis_name="s",
                                  num_cores=1)
    p1 = jax.ShapeDtypeStruct((VPAD,), jnp.float32)
    return pl.kernel(
        _sc_body,
        out_type=(p1, p1, p1),
        mesh=mesh,
        compiler_params=pltpu.CompilerParams(use_tc_tiling_on_sc=False,
                                             needs_layout_passes=False),
        scratch_types=(
            [pltpu.HBM((VPAD, TW), jnp.float32),
             pltpu.VMEM_SHARED((VPAD, AW), jnp.float32)]
            + [pltpu.VMEM((VC,), jnp.float32)] * 6
            + [pltpu.VMEM((VC, TW), jnp.float32),
               pltpu.VMEM((VC, AW), jnp.float32),
               pltpu.VMEM((CH, 128), jnp.int32),
               pltpu.VMEM((CH, 128, TW), jnp.float32),
               pltpu.VMEM((CH * 3, 128), jnp.float32),
               pltpu.VMEM((CH, 128, AW), jnp.float32),
               pltpu.SemaphoreType.DMA]),
    )(xh, yh, zh, mh, wh, ch, idxh, ixh)


def kernel(V_predict, L_last, V_w, V_mass_no_inf, C_shape, C_init_shape,
           V_compliance):
    f32 = jnp.float32
    vp = V_predict.astype(f32)

    def pad1(a):
        return jnp.pad(a, (0, VPAD - NUM_V))

    xh = pad1(vp[:, 0])
    yh = pad1(vp[:, 1])
    zh = pad1(vp[:, 2])
    mh = pad1(V_mass_no_inf.astype(f32)[:, 0])
    wh = pad1(V_w.astype(f32)[:, 0])
    ch = pad1(V_compliance.astype(f32)[:, 0])
    idxh = C_shape.astype(jnp.int32).reshape(NG * CH, 128)
    ixh = C_init_shape.astype(f32).reshape(NG * CH * 3, 128)
    ox, oy, oz = _sc_call(xh, yh, zh, mh, wh, ch, idxh, ixh)
    out = jnp.stack([ox[:NUM_V], oy[:NUM_V], oz[:NUM_V]], axis=1)
    return out.astype(V_predict.dtype), L_last


# speedup vs baseline: 4.2185x; 4.2185x over previous
import jax
import jax.numpy as jnp
from jax import lax
from jax.experimental import pallas as pl
from jax.experimental.pallas import tpu as pltpu
from jax.experimental.pallas import tpu_sc as plsc

NUM_V = 50000
NUM_C = 20000
P = 32

L = 16
NS = 16
GC = 16
SLOTS = GC * P
CH = SLOTS // 128
NG = NUM_C // GC
GPT = NG // NS
GCH = 20
TW = 16
AW = 8
VPAD = 51200
VT = VPAD // NS
VC = 800
CPAD = 20096


def _sc_body(xh, yh, zh, mh, wh, ch, idxn, initn,
             ox, oy, oz,
             table_hs, acc_s,
             pbx, pby, pbz, pbm, pbw, pbc, tb, ab,
             ib, xb, idx_f, rows_v, delta_v, sem):
    wid = lax.axis_index("s")
    lids = lax.iota(jnp.int32, L)
    cols = [jnp.full((L,), c, jnp.int32) for c in range(TW)]
    zero16 = jnp.zeros((L,), jnp.float32)
    o0 = wid * VT

    planes = (pbx, pby, pbz, pbm, pbw, pbc)

    for k in range(VT // VC):
        base = o0 + k * VC
        bsl = pl.ds(base, VC)
        for src, dst in zip((xh, yh, zh, mh, wh, ch), planes):
            pltpu.sync_copy(src.at[bsl], dst)

        def pack_body(i, carry):
            rows = i * jnp.int32(L) + lids
            isl = pl.ds(i * jnp.int32(L), L)
            for c in range(6):
                v = planes[c][isl]
                plsc.store_scatter(tb, [rows, cols[c]], v)
                if c < 3:
                    plsc.store_scatter(ab, [rows, cols[c]], v)
            for c in range(3, AW):
                plsc.store_scatter(ab, [rows, cols[c]], zero16)
            return carry

        lax.fori_loop(jnp.int32(0), jnp.int32(VC // L), pack_body,
                      jnp.int32(0))
        pltpu.sync_copy(tb, table_hs.at[bsl])
        pltpu.sync_copy(ab, acc_s.at[bsl])

    for q in range(SLOTS // L):
        sv = lids + q * L
        jv = lax.shift_right_logical(sv, jnp.int32(7))
        rv = lax.bitwise_and(sv, jnp.int32(127))
        for c in range(3, AW):
            plsc.store_scatter(delta_v, [jv, rv, cols[c]], zero16)

    plsc.subcore_barrier()

    n = jnp.int32(GPT) + jnp.where(wid < 2, jnp.int32(1), jnp.int32(0))
    s = wid * jnp.int32(GPT) + jnp.minimum(wid, jnp.int32(2))
    cb = s * jnp.int32(GC)

    for k in range(4):
        ck = cb + jnp.int32(k * GCH * GC)
        pltpu.sync_copy(initn.at[:, pl.ds(ck, GCH * GC)], ib)
        pltpu.sync_copy(idxn.at[:, pl.ds(ck, GCH * GC)], xb)

        def group_body(j, carry):
            g = jnp.int32(k * GCH) + j

            @pl.when(g < n)
            def _():
                co = j * jnp.int32(GC)
                for p in range(P):
                    idx_f[p // 8, pl.ds((p % 8) * L, L)] = xb[p, pl.ds(co, L)]

                descs = [pltpu.async_copy(table_hs.at[idx_f.at[jnp.int32(j2)]],
                                          rows_v.at[jnp.int32(j2)], sem)
                         for j2 in range(CH)]
                for d in descs:
                    d.wait()

                msum = zero16
                wx = zero16
                wy = zero16
                wz = zero16
                for p in range(P):
                    jp = cols[p // 8]
                    rvec = lids + (p % 8) * L
                    x = plsc.load_gather(rows_v, [jp, rvec, cols[0]])
                    y = plsc.load_gather(rows_v, [jp, rvec, cols[1]])
                    z = plsc.load_gather(rows_v, [jp, rvec, cols[2]])
                    m = plsc.load_gather(rows_v, [jp, rvec, cols[3]])
                    msum = msum + m
                    wx = wx + m * x
                    wy = wy + m * y
                    wz = wz + m * z
                cx = wx / msum
                cy = wy / msum
                cz = wz / msum

                for p in range(P):
                    jp = cols[p // 8]
                    rvec = lids + (p % 8) * L
                    x = plsc.load_gather(rows_v, [jp, rvec, cols[0]])
                    y = plsc.load_gather(rows_v, [jp, rvec, cols[1]])
                    z = plsc.load_gather(rows_v, [jp, rvec, cols[2]])
                    w = plsc.load_gather(rows_v, [jp, rvec, cols[4]])
                    cm = plsc.load_gather(rows_v, [jp, rvec, cols[5]])
                    csl = pl.ds(co, L)
                    ix = ib[p, csl]
                    iy = ib[P + p, csl]
                    iz = ib[2 * P + p, csl]
                    s2 = w / cm
                    plsc.store_scatter(delta_v, [jp, rvec, cols[0]],
                                       s2 * (ix - x + cx))
                    plsc.store_scatter(delta_v, [jp, rvec, cols[1]],
                                       s2 * (iy - y + cy))
                    plsc.store_scatter(delta_v, [jp, rvec, cols[2]],
                                       s2 * (iz - z + cz))

                for j2 in range(CH):
                    pltpu.sync_copy(delta_v.at[jnp.int32(j2)],
                                    acc_s.at[idx_f.at[jnp.int32(j2)]],
                                    add=True)

            return carry

        lax.fori_loop(jnp.int32(0), jnp.int32(GCH), group_body, jnp.int32(0))

    plsc.subcore_barrier()

    for k in range(VT // VC):
        base = o0 + k * VC
        bsl = pl.ds(base, VC)
        pltpu.sync_copy(acc_s.at[bsl], ab)

        def unpack_body(i, carry):
            rows = i * jnp.int32(L) + lids
            isl = pl.ds(i * jnp.int32(L), L)
            for c, pb in ((0, pbx), (1, pby), (2, pbz)):
                pb[isl] = plsc.load_gather(ab, [rows, cols[c]])
            return carry

        lax.fori_loop(jnp.int32(0), jnp.int32(VC // L), unpack_body,
                      jnp.int32(0))
        pltpu.sync_copy(pbx, ox.at[bsl])
        pltpu.sync_copy(pby, oy.at[bsl])
        pltpu.sync_copy(pbz, oz.at[bsl])


@jax.jit
def _sc_call(xh, yh, zh, mh, wh, ch, idxn, initn):
    mesh = plsc.VectorSubcoreMesh(core_axis_name="c", subcore_axis_name="s",
                                  num_cores=1)
    p1 = jax.ShapeDtypeStruct((VPAD,), jnp.float32)
    return pl.kernel(
        _sc_body,
        out_type=(p1, p1, p1),
        mesh=mesh,
        compiler_params=pltpu.CompilerParams(use_tc_tiling_on_sc=False,
                                             needs_layout_passes=False),
        scratch_types=(
            [pltpu.HBM((VPAD, TW), jnp.float32),
             pltpu.VMEM_SHARED((VPAD, AW), jnp.float32)]
            + [pltpu.VMEM((VC,), jnp.float32)] * 6
            + [pltpu.VMEM((VC, TW), jnp.float32),
               pltpu.VMEM((VC, AW), jnp.float32),
               pltpu.VMEM((3 * P, GCH * GC), jnp.float32),
               pltpu.VMEM((P, GCH * GC), jnp.int32),
               pltpu.VMEM((CH, 128), jnp.int32),
               pltpu.VMEM((CH, 128, TW), jnp.float32),
               pltpu.VMEM((CH, 128, AW), jnp.float32),
               pltpu.SemaphoreType.DMA]),
    )(xh, yh, zh, mh, wh, ch, idxn, initn)


def kernel(V_predict, L_last, V_w, V_mass_no_inf, C_shape, C_init_shape,
           V_compliance):
    f32 = jnp.float32
    vp = V_predict.astype(f32)

    def pad1(a):
        return jnp.pad(a, (0, VPAD - NUM_V))

    xh = pad1(vp[:, 0])
    yh = pad1(vp[:, 1])
    zh = pad1(vp[:, 2])
    mh = pad1(V_mass_no_inf.astype(f32)[:, 0])
    wh = pad1(V_w.astype(f32)[:, 0])
    ch = pad1(V_compliance.astype(f32)[:, 0])
    idxn = jnp.pad(C_shape.transpose(1, 0).astype(jnp.int32),
                   ((0, 0), (0, CPAD - NUM_C)))
    initn = jnp.pad(C_init_shape.astype(f32).transpose(2, 1, 0)
                    .reshape(3 * P, NUM_C),
                    ((0, 0), (0, CPAD - NUM_C)))
    ox, oy, oz = _sc_call(xh, yh, zh, mh, wh, ch, idxn, initn)
    out = jnp.stack([ox[:NUM_V], oy[:NUM_V], oz[:NUM_V]], axis=1)
    return out.astype(V_predict.dtype), L_last


# speedup vs baseline: 6.3366x; 1.5021x over previous
import jax
import jax.numpy as jnp
from jax import lax
from jax.experimental import pallas as pl
from jax.experimental.pallas import tpu as pltpu
from jax.experimental.pallas import tpu_sc as plsc

NUM_V = 50000
NUM_C = 20000
P = 32

L = 16
NS = 16
GC = 16
SLOTS = GC * P
CH = SLOTS // 128
NG = NUM_C // GC
NW = 2 * NS
GPW = NG // NW
GCH = 20
TW = 16
AW = 8
VPAD = 51200
VT = VPAD // NS
VC = 800
CPAD = 20096


def _sc_body(xh, yh, zh, mh, wh, ch, idxn, initn,
             ox0, oy0, oz0, ox1, oy1, oz1,
             table_hs, acc_s,
             pbx, pby, pbz, pbm, pbw, pbc, tb, ab,
             ib, xb, idx_f, rows_v, delta_v, sem):
    cid = lax.axis_index("c")
    sid = lax.axis_index("s")
    gwid = cid * jnp.int32(NS) + sid
    lids = lax.iota(jnp.int32, L)
    cols = [jnp.full((L,), c, jnp.int32) for c in range(TW)]
    zero16 = jnp.zeros((L,), jnp.float32)
    core0 = cid == 0
    o0 = sid * VT

    planes = (pbx, pby, pbz, pbm, pbw, pbc)

    for k in range(VT // VC):
        base = o0 + k * VC
        bsl = pl.ds(base, VC)
        for src, dst in zip((xh, yh, zh, mh, wh, ch), planes):
            pltpu.sync_copy(src.at[bsl], dst)

        def pack_body(i, carry):
            rows = i * jnp.int32(L) + lids
            isl = pl.ds(i * jnp.int32(L), L)
            for c in range(6):
                v = planes[c][isl]
                plsc.store_scatter(tb, [rows, cols[c]], v)
                if c < 3:
                    plsc.store_scatter(ab, [rows, cols[c]],
                                       jnp.where(core0, v, zero16))
            for c in range(3, AW):
                plsc.store_scatter(ab, [rows, cols[c]], zero16)
            return carry

        lax.fori_loop(jnp.int32(0), jnp.int32(VC // L), pack_body,
                      jnp.int32(0))
        pltpu.sync_copy(tb, table_hs.at[bsl])
        pltpu.sync_copy(ab, acc_s.at[bsl])

    for q in range(SLOTS // L):
        sv = lids + q * L
        jv = lax.shift_right_logical(sv, jnp.int32(7))
        rv = lax.bitwise_and(sv, jnp.int32(127))
        for c in range(3, AW):
            plsc.store_scatter(delta_v, [jv, rv, cols[c]], zero16)

    plsc.subcore_barrier()

    n = jnp.int32(GPW) + jnp.where(gwid < 2, jnp.int32(1), jnp.int32(0))
    s = gwid * jnp.int32(GPW) + jnp.minimum(gwid, jnp.int32(2))
    cb = s * jnp.int32(GC)

    for k in range(2):
        ck = cb + jnp.int32(k * GCH * GC)
        pltpu.sync_copy(initn.at[:, pl.ds(ck, GCH * GC)], ib)
        pltpu.sync_copy(idxn.at[:, pl.ds(ck, GCH * GC)], xb)

        def group_body(j, carry):
            g = jnp.int32(k * GCH) + j

            @pl.when(g < n)
            def _():
                co = j * jnp.int32(GC)
                for p in range(P):
                    idx_f[p // 8, pl.ds((p % 8) * L, L)] = xb[p, pl.ds(co, L)]

                descs = [pltpu.async_copy(table_hs.at[idx_f.at[jnp.int32(j2)]],
                                          rows_v.at[jnp.int32(j2)], sem)
                         for j2 in range(CH)]
                for d in descs:
                    d.wait()

                msum = zero16
                wx = zero16
                wy = zero16
                wz = zero16
                for p in range(P):
                    jp = cols[p // 8]
                    rvec = lids + (p % 8) * L
                    x = plsc.load_gather(rows_v, [jp, rvec, cols[0]])
                    y = plsc.load_gather(rows_v, [jp, rvec, cols[1]])
                    z = plsc.load_gather(rows_v, [jp, rvec, cols[2]])
                    m = plsc.load_gather(rows_v, [jp, rvec, cols[3]])
                    msum = msum + m
                    wx = wx + m * x
                    wy = wy + m * y
                    wz = wz + m * z
                cx = wx / msum
                cy = wy / msum
                cz = wz / msum

                for p in range(P):
                    jp = cols[p // 8]
                    rvec = lids + (p % 8) * L
                    x = plsc.load_gather(rows_v, [jp, rvec, cols[0]])
                    y = plsc.load_gather(rows_v, [jp, rvec, cols[1]])
                    z = plsc.load_gather(rows_v, [jp, rvec, cols[2]])
                    w = plsc.load_gather(rows_v, [jp, rvec, cols[4]])
                    cm = plsc.load_gather(rows_v, [jp, rvec, cols[5]])
                    csl = pl.ds(co, L)
                    ix = ib[p, csl]
                    iy = ib[P + p, csl]
                    iz = ib[2 * P + p, csl]
                    s2 = w / cm
                    plsc.store_scatter(delta_v, [jp, rvec, cols[0]],
                                       s2 * (ix - x + cx))
                    plsc.store_scatter(delta_v, [jp, rvec, cols[1]],
                                       s2 * (iy - y + cy))
                    plsc.store_scatter(delta_v, [jp, rvec, cols[2]],
                                       s2 * (iz - z + cz))

                for j2 in range(CH):
                    pltpu.sync_copy(delta_v.at[jnp.int32(j2)],
                                    acc_s.at[idx_f.at[jnp.int32(j2)]],
                                    add=True)

            return carry

        lax.fori_loop(jnp.int32(0), jnp.int32(GCH), group_body, jnp.int32(0))

    plsc.subcore_barrier()

    for k in range(VT // VC):
        base = o0 + k * VC
        bsl = pl.ds(base, VC)
        pltpu.sync_copy(acc_s.at[bsl], ab)

        def unpack_body(i, carry):
            rows = i * jnp.int32(L) + lids
            isl = pl.ds(i * jnp.int32(L), L)
            for c, pb in ((0, pbx), (1, pby), (2, pbz)):
                pb[isl] = plsc.load_gather(ab, [rows, cols[c]])
            return carry

        lax.fori_loop(jnp.int32(0), jnp.int32(VC // L), unpack_body,
                      jnp.int32(0))

        @pl.when(core0)
        def _():
            pltpu.sync_copy(pbx, ox0.at[bsl])
            pltpu.sync_copy(pby, oy0.at[bsl])
            pltpu.sync_copy(pbz, oz0.at[bsl])

        @pl.when(jnp.logical_not(core0))
        def _():
            pltpu.sync_copy(pbx, ox1.at[bsl])
            pltpu.sync_copy(pby, oy1.at[bsl])
            pltpu.sync_copy(pbz, oz1.at[bsl])


@jax.jit
def _sc_call(xh, yh, zh, mh, wh, ch, idxn, initn):
    mesh = plsc.VectorSubcoreMesh(core_axis_name="c", subcore_axis_name="s",
                                  num_cores=2)
    p1 = jax.ShapeDtypeStruct((VPAD,), jnp.float32)
    return pl.kernel(
        _sc_body,
        out_type=(p1, p1, p1, p1, p1, p1),
        mesh=mesh,
        compiler_params=pltpu.CompilerParams(use_tc_tiling_on_sc=False,
                                             needs_layout_passes=False),
        scratch_types=(
            [pltpu.HBM((VPAD, TW), jnp.float32),
             pltpu.VMEM_SHARED((VPAD, AW), jnp.float32)]
            + [pltpu.VMEM((VC,), jnp.float32)] * 6
            + [pltpu.VMEM((VC, TW), jnp.float32),
               pltpu.VMEM((VC, AW), jnp.float32),
               pltpu.VMEM((3 * P, GCH * GC), jnp.float32),
               pltpu.VMEM((P, GCH * GC), jnp.int32),
               pltpu.VMEM((CH, 128), jnp.int32),
               pltpu.VMEM((CH, 128, TW), jnp.float32),
               pltpu.VMEM((CH, 128, AW), jnp.float32),
               pltpu.SemaphoreType.DMA]),
    )(xh, yh, zh, mh, wh, ch, idxn, initn)


def kernel(V_predict, L_last, V_w, V_mass_no_inf, C_shape, C_init_shape,
           V_compliance):
    f32 = jnp.float32
    vp = V_predict.astype(f32)

    def pad1(a):
        return jnp.pad(a, (0, VPAD - NUM_V))

    xh = pad1(vp[:, 0])
    yh = pad1(vp[:, 1])
    zh = pad1(vp[:, 2])
    mh = pad1(V_mass_no_inf.astype(f32)[:, 0])
    wh = pad1(V_w.astype(f32)[:, 0])
    ch = pad1(V_compliance.astype(f32)[:, 0])
    idxn = jnp.pad(C_shape.transpose(1, 0).astype(jnp.int32),
                   ((0, 0), (0, CPAD - NUM_C)))
    initn = jnp.pad(C_init_shape.astype(f32).transpose(2, 1, 0)
                    .reshape(3 * P, NUM_C),
                    ((0, 0), (0, CPAD - NUM_C)))
    ox0, oy0, oz0, ox1, oy1, oz1 = _sc_call(xh, yh, zh, mh, wh, ch,
                                            idxn, initn)
    out = jnp.stack([(ox0 + ox1)[:NUM_V], (oy0 + oy1)[:NUM_V],
                     (oz0 + oz1)[:NUM_V]], axis=1)
    return out.astype(V_predict.dtype), L_last


# speedup vs baseline: 7.2193x; 1.1393x over previous
import jax
import jax.numpy as jnp
from jax import lax
from jax.experimental import pallas as pl
from jax.experimental.pallas import tpu as pltpu
from jax.experimental.pallas import tpu_sc as plsc

NUM_V = 50000
NUM_C = 20000
P = 32

L = 16
NS = 16
GC = 16
SLOTS = GC * P
CH = SLOTS // 128
NG = NUM_C // GC
NW = 2 * NS
GPW = NG // NW
GCH = 20
TW = 16
AW = 8
VPAD = 51200
VT = VPAD // NS
VC = 800
CPAD = 20096


def _sc_body(xh, yh, zh, mh, wh, ch, idxn, initn,
             ox0, oy0, oz0, ox1, oy1, oz1,
             table_hs, acc_s,
             pbx, pby, pbz, pbm, pbw, pbc, tb, ab,
             ib, xb, idx_fa, idx_fb, rows_va, rows_vb, dummy_h,
             delta_v, sem):
    cid = lax.axis_index("c")
    sid = lax.axis_index("s")
    gwid = cid * jnp.int32(NS) + sid
    lids = lax.iota(jnp.int32, L)
    cols = [jnp.full((L,), c, jnp.int32) for c in range(TW)]
    zero16 = jnp.zeros((L,), jnp.float32)
    core0 = cid == 0
    o0 = sid * VT

    planes = (pbx, pby, pbz, pbm, pbw, pbc)

    for k in range(VT // VC):
        base = o0 + k * VC
        bsl = pl.ds(base, VC)
        for src, dst in zip((xh, yh, zh, mh, wh, ch), planes):
            pltpu.sync_copy(src.at[bsl], dst)

        def pack_body(i, carry):
            rows = i * jnp.int32(L) + lids
            isl = pl.ds(i * jnp.int32(L), L)
            for c in range(6):
                v = planes[c][isl]
                plsc.store_scatter(tb, [rows, cols[c]], v)
                if c < 3:
                    plsc.store_scatter(ab, [rows, cols[c]],
                                       jnp.where(core0, v, zero16))
            for c in range(3, AW):
                plsc.store_scatter(ab, [rows, cols[c]], zero16)
            return carry

        lax.fori_loop(jnp.int32(0), jnp.int32(VC // L), pack_body,
                      jnp.int32(0))
        pltpu.sync_copy(tb, table_hs.at[bsl])
        pltpu.sync_copy(ab, acc_s.at[bsl])

    for q in range(SLOTS // L):
        sv = lids + q * L
        jv = lax.shift_right_logical(sv, jnp.int32(7))
        rv = lax.bitwise_and(sv, jnp.int32(127))
        for c in range(3, AW):
            plsc.store_scatter(delta_v, [jv, rv, cols[c]], zero16)

    plsc.subcore_barrier()

    n = jnp.int32(GPW) + jnp.where(gwid < 2, jnp.int32(1), jnp.int32(0))
    s = gwid * jnp.int32(GPW) + jnp.minimum(gwid, jnp.int32(2))
    cb = s * jnp.int32(GC)

    idx_bufs = (idx_fa, idx_fb)
    row_bufs = (rows_va, rows_vb)

    def transcribe_and_fire(j, idx_f, rows_v):
        co = j * jnp.int32(GC)
        for p in range(P):
            idx_f[p // 8, pl.ds((p % 8) * L, L)] = xb[p, pl.ds(co, L)]
        for j2 in range(CH):
            pltpu.async_copy(table_hs.at[idx_f.at[jnp.int32(j2)]],
                             rows_v.at[jnp.int32(j2)], sem)

    for k in range(2):
        ck = cb + jnp.int32(k * GCH * GC)
        pltpu.sync_copy(initn.at[:, pl.ds(ck, GCH * GC)], ib)
        pltpu.sync_copy(idxn.at[:, pl.ds(ck, GCH * GC)], xb)
        ln = jnp.minimum(n - jnp.int32(k * GCH), jnp.int32(GCH))

        @pl.when(jnp.int32(0) < ln)
        def _():
            transcribe_and_fire(jnp.int32(0), idx_fa, rows_va)

        def pair_body(jj, carry):
            for u in range(2):
                j = jj * jnp.int32(2) + u
                idx_f = idx_bufs[u]
                rows_v = row_bufs[u]

                @pl.when(j < ln)
                def _():
                    pltpu.make_async_copy(dummy_h, rows_v, sem).wait()

                    @pl.when(j + 1 < ln)
                    def _():
                        transcribe_and_fire(j + 1, idx_bufs[1 - u],
                                            row_bufs[1 - u])

                    co = j * jnp.int32(GC)

                    msum = zero16
                    wx = zero16
                    wy = zero16
                    wz = zero16
                    for p in range(P):
                        jp = cols[p // 8]
                        rvec = lids + (p % 8) * L
                        x = plsc.load_gather(rows_v, [jp, rvec, cols[0]])
                        y = plsc.load_gather(rows_v, [jp, rvec, cols[1]])
                        z = plsc.load_gather(rows_v, [jp, rvec, cols[2]])
                        m = plsc.load_gather(rows_v, [jp, rvec, cols[3]])
                        msum = msum + m
                        wx = wx + m * x
                        wy = wy + m * y
                        wz = wz + m * z
                    cx = wx / msum
                    cy = wy / msum
                    cz = wz / msum

                    for p in range(P):
                        jp = cols[p // 8]
                        rvec = lids + (p % 8) * L
                        x = plsc.load_gather(rows_v, [jp, rvec, cols[0]])
                        y = plsc.load_gather(rows_v, [jp, rvec, cols[1]])
                        z = plsc.load_gather(rows_v, [jp, rvec, cols[2]])
                        w = plsc.load_gather(rows_v, [jp, rvec, cols[4]])
                        cm = plsc.load_gather(rows_v, [jp, rvec, cols[5]])
                        csl = pl.ds(co, L)
                        ix = ib[p, csl]
                        iy = ib[P + p, csl]
                        iz = ib[2 * P + p, csl]
                        s2 = w / cm
                        plsc.store_scatter(delta_v, [jp, rvec, cols[0]],
                                           s2 * (ix - x + cx))
                        plsc.store_scatter(delta_v, [jp, rvec, cols[1]],
                                           s2 * (iy - y + cy))
                        plsc.store_scatter(delta_v, [jp, rvec, cols[2]],
                                           s2 * (iz - z + cz))

                    for j2 in range(CH):
                        pltpu.sync_copy(delta_v.at[jnp.int32(j2)],
                                        acc_s.at[idx_f.at[jnp.int32(j2)]],
                                        add=True)

            return carry

        lax.fori_loop(jnp.int32(0), jnp.int32(GCH // 2), pair_body,
                      jnp.int32(0))

    plsc.subcore_barrier()

    for k in range(VT // VC):
        base = o0 + k * VC
        bsl = pl.ds(base, VC)
        pltpu.sync_copy(acc_s.at[bsl], ab)

        def unpack_body(i, carry):
            rows = i * jnp.int32(L) + lids
            isl = pl.ds(i * jnp.int32(L), L)
            for c, pb in ((0, pbx), (1, pby), (2, pbz)):
                pb[isl] = plsc.load_gather(ab, [rows, cols[c]])
            return carry

        lax.fori_loop(jnp.int32(0), jnp.int32(VC // L), unpack_body,
                      jnp.int32(0))

        @pl.when(core0)
        def _():
            pltpu.sync_copy(pbx, ox0.at[bsl])
            pltpu.sync_copy(pby, oy0.at[bsl])
            pltpu.sync_copy(pbz, oz0.at[bsl])

        @pl.when(jnp.logical_not(core0))
        def _():
            pltpu.sync_copy(pbx, ox1.at[bsl])
            pltpu.sync_copy(pby, oy1.at[bsl])
            pltpu.sync_copy(pbz, oz1.at[bsl])


@jax.jit
def _sc_call(xh, yh, zh, mh, wh, ch, idxn, initn):
    mesh = plsc.VectorSubcoreMesh(core_axis_name="c", subcore_axis_name="s",
                                  num_cores=2)
    p1 = jax.ShapeDtypeStruct((VPAD,), jnp.float32)
    return pl.kernel(
        _sc_body,
        out_type=(p1, p1, p1, p1, p1, p1),
        mesh=mesh,
        compiler_params=pltpu.CompilerParams(use_tc_tiling_on_sc=False,
                                             needs_layout_passes=False),
        scratch_types=(
            [pltpu.HBM((VPAD, TW), jnp.float32),
             pltpu.VMEM_SHARED((VPAD, AW), jnp.float32)]
            + [pltpu.VMEM((VC,), jnp.float32)] * 6
            + [pltpu.VMEM((VC, TW), jnp.float32),
               pltpu.VMEM((VC, AW), jnp.float32),
               pltpu.VMEM((3 * P, GCH * GC), jnp.float32),
               pltpu.VMEM((P, GCH * GC), jnp.int32),
               pltpu.VMEM((CH, 128), jnp.int32),
               pltpu.VMEM((CH, 128), jnp.int32),
               pltpu.VMEM((CH, 128, TW), jnp.float32),
               pltpu.VMEM((CH, 128, TW), jnp.float32),
               pltpu.HBM((CH, 128, TW), jnp.float32),
               pltpu.VMEM((CH, 128, AW), jnp.float32),
               pltpu.SemaphoreType.DMA]),
    )(xh, yh, zh, mh, wh, ch, idxn, initn)


def kernel(V_predict, L_last, V_w, V_mass_no_inf, C_shape, C_init_shape,
           V_compliance):
    f32 = jnp.float32
    vp = V_predict.astype(f32)

    def pad1(a):
        return jnp.pad(a, (0, VPAD - NUM_V))

    xh = pad1(vp[:, 0])
    yh = pad1(vp[:, 1])
    zh = pad1(vp[:, 2])
    mh = pad1(V_mass_no_inf.astype(f32)[:, 0])
    wh = pad1(V_w.astype(f32)[:, 0])
    ch = pad1(V_compliance.astype(f32)[:, 0])
    idxn = jnp.pad(C_shape.transpose(1, 0).astype(jnp.int32),
                   ((0, 0), (0, CPAD - NUM_C)))
    initn = jnp.pad(C_init_shape.astype(f32).transpose(2, 1, 0)
                    .reshape(3 * P, NUM_C),
                    ((0, 0), (0, CPAD - NUM_C)))
    ox0, oy0, oz0, ox1, oy1, oz1 = _sc_call(xh, yh, zh, mh, wh, ch,
                                            idxn, initn)
    out = jnp.stack([(ox0 + ox1)[:NUM_V], (oy0 + oy1)[:NUM_V],
                     (oz0 + oz1)[:NUM_V]], axis=1)
    return out.astype(V_predict.dtype), L_last


# speedup vs baseline: 7.6837x; 1.0643x over previous
import jax
import jax.numpy as jnp
from jax import lax
from jax.experimental import pallas as pl
from jax.experimental.pallas import tpu as pltpu
from jax.experimental.pallas import tpu_sc as plsc

NUM_V = 50000
NUM_C = 20000
P = 32

L = 16
NS = 16
GC = 16
SLOTS = GC * P
CH = SLOTS // 128
NG = NUM_C // GC
NW = 2 * NS
GPW = NG // NW
GCH = 20
TW = 16
AW = 8
VPAD = 51200
VT = VPAD // NS
VC = 800
CPAD = 20096


def _sc_body(xh, yh, zh, mh, wh, ch, idxn, initn,
             ox0, oy0, oz0, ox1, oy1, oz1,
             table_hs, acc_s,
             pbx, pby, pbz, pbm, pbw, pbc, tb, ab,
             ib, xb, idx_fa, idx_fb, rows_va, rows_vb, dummy_h,
             delta_va, delta_vb, dummy_h2, sem, sem2):
    cid = lax.axis_index("c")
    sid = lax.axis_index("s")
    gwid = cid * jnp.int32(NS) + sid
    lids = lax.iota(jnp.int32, L)
    cols = [jnp.full((L,), c, jnp.int32) for c in range(TW)]
    zero16 = jnp.zeros((L,), jnp.float32)
    core0 = cid == 0
    o0 = sid * VT

    planes = (pbx, pby, pbz, pbm, pbw, pbc)

    for k in range(VT // VC):
        base = o0 + k * VC
        bsl = pl.ds(base, VC)
        for src, dst in zip((xh, yh, zh, mh, wh, ch), planes):
            pltpu.sync_copy(src.at[bsl], dst)

        def pack_body(i, carry):
            rows = i * jnp.int32(L) + lids
            isl = pl.ds(i * jnp.int32(L), L)
            for c in range(6):
                v = planes[c][isl]
                plsc.store_scatter(tb, [rows, cols[c]], v)
                if c < 3:
                    plsc.store_scatter(ab, [rows, cols[c]],
                                       jnp.where(core0, v, zero16))
            for c in range(3, AW):
                plsc.store_scatter(ab, [rows, cols[c]], zero16)
            return carry

        lax.fori_loop(jnp.int32(0), jnp.int32(VC // L), pack_body,
                      jnp.int32(0))
        pltpu.sync_copy(tb, table_hs.at[bsl])
        pltpu.sync_copy(ab, acc_s.at[bsl])

    for q in range(SLOTS // L):
        sv = lids + q * L
        jv = lax.shift_right_logical(sv, jnp.int32(7))
        rv = lax.bitwise_and(sv, jnp.int32(127))
        for c in range(3, AW):
            plsc.store_scatter(delta_va, [jv, rv, cols[c]], zero16)
            plsc.store_scatter(delta_vb, [jv, rv, cols[c]], zero16)

    plsc.subcore_barrier()

    n = jnp.int32(GPW) + jnp.where(gwid < 2, jnp.int32(1), jnp.int32(0))
    s = gwid * jnp.int32(GPW) + jnp.minimum(gwid, jnp.int32(2))
    cb = s * jnp.int32(GC)

    idx_bufs = (idx_fa, idx_fb)
    row_bufs = (rows_va, rows_vb)
    delta_bufs = (delta_va, delta_vb)

    def transcribe_and_fire(j, idx_f, rows_v):
        co = j * jnp.int32(GC)
        for p in range(P):
            idx_f[p // 8, pl.ds((p % 8) * L, L)] = xb[p, pl.ds(co, L)]
        for j2 in range(CH):
            pltpu.async_copy(table_hs.at[idx_f.at[jnp.int32(j2)]],
                             rows_v.at[jnp.int32(j2)], sem)

    for k in range(2):
        ck = cb + jnp.int32(k * GCH * GC)
        pltpu.sync_copy(initn.at[:, pl.ds(ck, GCH * GC)], ib)
        pltpu.sync_copy(idxn.at[:, pl.ds(ck, GCH * GC)], xb)
        ln = jnp.minimum(n - jnp.int32(k * GCH), jnp.int32(GCH))

        @pl.when(jnp.int32(0) < ln)
        def _():
            transcribe_and_fire(jnp.int32(0), idx_fa, rows_va)

        def pair_body(jj, carry):
            for u in range(2):
                j = jj * jnp.int32(2) + u
                idx_f = idx_bufs[u]
                rows_v = row_bufs[u]
                delta_v = delta_bufs[u]

                @pl.when(j < ln)
                def _():
                    pltpu.make_async_copy(dummy_h, rows_v, sem).wait()

                    @pl.when(j >= 2)
                    def _():
                        pltpu.make_async_copy(dummy_h2, delta_v, sem2).wait()

                    @pl.when(j + 1 < ln)
                    def _():
                        transcribe_and_fire(j + 1, idx_bufs[1 - u],
                                            row_bufs[1 - u])

                    co = j * jnp.int32(GC)

                    msum = zero16
                    wx = zero16
                    wy = zero16
                    wz = zero16
                    for p in range(P):
                        jp = cols[p // 8]
                        rvec = lids + (p % 8) * L
                        x = plsc.load_gather(rows_v, [jp, rvec, cols[0]])
                        y = plsc.load_gather(rows_v, [jp, rvec, cols[1]])
                        z = plsc.load_gather(rows_v, [jp, rvec, cols[2]])
                        m = plsc.load_gather(rows_v, [jp, rvec, cols[3]])
                        msum = msum + m
                        wx = wx + m * x
                        wy = wy + m * y
                        wz = wz + m * z
                    cx = wx / msum
                    cy = wy / msum
                    cz = wz / msum

                    for p in range(P):
                        jp = cols[p // 8]
                        rvec = lids + (p % 8) * L
                        x = plsc.load_gather(rows_v, [jp, rvec, cols[0]])
                        y = plsc.load_gather(rows_v, [jp, rvec, cols[1]])
                        z = plsc.load_gather(rows_v, [jp, rvec, cols[2]])
                        w = plsc.load_gather(rows_v, [jp, rvec, cols[4]])
                        cm = plsc.load_gather(rows_v, [jp, rvec, cols[5]])
                        csl = pl.ds(co, L)
                        ix = ib[p, csl]
                        iy = ib[P + p, csl]
                        iz = ib[2 * P + p, csl]
                        s2 = w / cm
                        plsc.store_scatter(delta_v, [jp, rvec, cols[0]],
                                           s2 * (ix - x + cx))
                        plsc.store_scatter(delta_v, [jp, rvec, cols[1]],
                                           s2 * (iy - y + cy))
                        plsc.store_scatter(delta_v, [jp, rvec, cols[2]],
                                           s2 * (iz - z + cz))

                    for j2 in range(CH):
                        pltpu.async_copy(delta_v.at[jnp.int32(j2)],
                                         acc_s.at[idx_f.at[jnp.int32(j2)]],
                                         sem2, add=True)

            return carry

        lax.fori_loop(jnp.int32(0), jnp.int32(GCH // 2), pair_body,
                      jnp.int32(0))
        pltpu.make_async_copy(dummy_h2, delta_va, sem2).wait()
        pltpu.make_async_copy(dummy_h2, delta_vb, sem2).wait()

    plsc.subcore_barrier()

    for k in range(VT // VC):
        base = o0 + k * VC
        bsl = pl.ds(base, VC)
        pltpu.sync_copy(acc_s.at[bsl], ab)

        def unpack_body(i, carry):
            rows = i * jnp.int32(L) + lids
            isl = pl.ds(i * jnp.int32(L), L)
            for c, pb in ((0, pbx), (1, pby), (2, pbz)):
                pb[isl] = plsc.load_gather(ab, [rows, cols[c]])
            return carry

        lax.fori_loop(jnp.int32(0), jnp.int32(VC // L), unpack_body,
                      jnp.int32(0))

        @pl.when(core0)
        def _():
            pltpu.sync_copy(pbx, ox0.at[bsl])
            pltpu.sync_copy(pby, oy0.at[bsl])
            pltpu.sync_copy(pbz, oz0.at[bsl])

        @pl.when(jnp.logical_not(core0))
        def _():
            pltpu.sync_copy(pbx, ox1.at[bsl])
            pltpu.sync_copy(pby, oy1.at[bsl])
            pltpu.sync_copy(pbz, oz1.at[bsl])


@jax.jit
def _sc_call(xh, yh, zh, mh, wh, ch, idxn, initn):
    mesh = plsc.VectorSubcoreMesh(core_axis_name="c", subcore_axis_name="s",
                                  num_cores=2)
    p1 = jax.ShapeDtypeStruct((VPAD,), jnp.float32)
    return pl.kernel(
        _sc_body,
        out_type=(p1, p1, p1, p1, p1, p1),
        mesh=mesh,
        compiler_params=pltpu.CompilerParams(use_tc_tiling_on_sc=False,
                                             needs_layout_passes=False),
        scratch_types=(
            [pltpu.HBM((VPAD, TW), jnp.float32),
             pltpu.VMEM_SHARED((VPAD, AW), jnp.float32)]
            + [pltpu.VMEM((VC,), jnp.float32)] * 6
            + [pltpu.VMEM((VC, TW), jnp.float32),
               pltpu.VMEM((VC, AW), jnp.float32),
               pltpu.VMEM((3 * P, GCH * GC), jnp.float32),
               pltpu.VMEM((P, GCH * GC), jnp.int32),
               pltpu.VMEM((CH, 128), jnp.int32),
               pltpu.VMEM((CH, 128), jnp.int32),
               pltpu.VMEM((CH, 128, TW), jnp.float32),
               pltpu.VMEM((CH, 128, TW), jnp.float32),
               pltpu.HBM((CH, 128, TW), jnp.float32),
               pltpu.VMEM((CH, 128, AW), jnp.float32),
               pltpu.VMEM((CH, 128, AW), jnp.float32),
               pltpu.HBM((CH, 128, AW), jnp.float32),
               pltpu.SemaphoreType.DMA,
               pltpu.SemaphoreType.DMA]),
    )(xh, yh, zh, mh, wh, ch, idxn, initn)


def kernel(V_predict, L_last, V_w, V_mass_no_inf, C_shape, C_init_shape,
           V_compliance):
    f32 = jnp.float32
    vp = V_predict.astype(f32)

    def pad1(a):
        return jnp.pad(a, (0, VPAD - NUM_V))

    xh = pad1(vp[:, 0])
    yh = pad1(vp[:, 1])
    zh = pad1(vp[:, 2])
    mh = pad1(V_mass_no_inf.astype(f32)[:, 0])
    wh = pad1(V_w.astype(f32)[:, 0])
    ch = pad1(V_compliance.astype(f32)[:, 0])
    idxn = jnp.pad(C_shape.transpose(1, 0).astype(jnp.int32),
                   ((0, 0), (0, CPAD - NUM_C)))
    initn = jnp.pad(C_init_shape.astype(f32).transpose(2, 1, 0)
                    .reshape(3 * P, NUM_C),
                    ((0, 0), (0, CPAD - NUM_C)))
    ox0, oy0, oz0, ox1, oy1, oz1 = _sc_call(xh, yh, zh, mh, wh, ch,
                                            idxn, initn)
    out = jnp.stack([(ox0 + ox1)[:NUM_V], (oy0 + oy1)[:NUM_V],
                     (oz0 + oz1)[:NUM_V]], axis=1)
    return out.astype(V_predict.dtype), L_last
